# Initial kernel scaffold; baseline (speedup 1.0000x reference)
import functools

import jax
import jax.numpy as jnp
from jax import lax
from jax.experimental import pallas as pl
from jax.experimental.pallas import tpu as pltpu
from jax.experimental.pallas import tpu_sc as plsc

N = 10000
D = 128
E = 320000

NC = 2
NS = 16
NW = NC * NS
L = 16

C = 80
NCH = E // C
CH_W = NCH // NW
ROWS_T = N // NS
ZB = 125

_mesh = plsc.VectorSubcoreMesh(
    core_axis_name="c", subcore_axis_name="s", num_cores=NC, num_subcores=NS
)


@functools.partial(
    pl.kernel,
    out_type=jax.ShapeDtypeStruct((NC, N, D), jnp.float32),
    mesh=_mesh,
    scratch_types=dict(
        acc=pltpu.VMEM_SHARED((N, D), jnp.float32),
        idx_s=pltpu.VMEM((CH_W, C), jnp.int32),
        idx_d=pltpu.VMEM((CH_W, C), jnp.int32),
        rows=pltpu.VMEM((C, D), jnp.float32),
        zbuf=pltpu.VMEM((ZB, D), jnp.float32),
        sem=pltpu.SemaphoreType.DMA,
    ),
)
def _agg_kernel(x_hbm, src_hbm, dst_hbm, out_hbm, *, acc, idx_s, idx_d, rows, zbuf, sem):
    c = lax.axis_index("c")
    s = lax.axis_index("s")
    wid = s * NC + c

    zeros16 = jnp.zeros((L,), jnp.float32)

    @pl.loop(0, ZB)
    def _zero(i):
        for q in range(D // L):
            zbuf[i, pl.ds(q * L, L)] = zeros16

    for r in range(ROWS_T // ZB):
        pltpu.sync_copy(zbuf, acc.at[pl.ds(s * ROWS_T + r * ZB, ZB)])
    plsc.subcore_barrier()

    pltpu.sync_copy(src_hbm.at[pl.ds(wid * CH_W, CH_W)], idx_s)
    pltpu.sync_copy(dst_hbm.at[pl.ds(wid * CH_W, CH_W)], idx_d)

    @pl.loop(0, CH_W)
    def _chunk(j):
        pltpu.async_copy(x_hbm.at[idx_s.at[j]], rows, sem).wait()
        pltpu.sync_copy(rows, acc.at[idx_d.at[j]], add=True)

    plsc.subcore_barrier()
    pltpu.sync_copy(
        acc.at[pl.ds(s * ROWS_T, ROWS_T)], out_hbm.at[c, pl.ds(s * ROWS_T, ROWS_T)]
    )


@functools.partial(
    pl.kernel,
    out_type=jax.ShapeDtypeStruct((NCH, C), jnp.float32),
    mesh=_mesh,
    scratch_types=dict(
        idx_a=pltpu.VMEM((CH_W, C), jnp.int32),
        idx_b=pltpu.VMEM((CH_W, C), jnp.int32),
        za=pltpu.VMEM((C, D), jnp.float32),
        zb=pltpu.VMEM((C, D), jnp.float32),
        lg=pltpu.VMEM((CH_W, C), jnp.float32),
        sem=pltpu.SemaphoreType.DMA,
    ),
)
def _decode_kernel(z_hbm, ps_hbm, pd_hbm, out_hbm, *, idx_a, idx_b, za, zb, lg, sem):
    c = lax.axis_index("c")
    s = lax.axis_index("s")
    wid = s * NC + c

    pltpu.sync_copy(ps_hbm.at[pl.ds(wid * CH_W, CH_W)], idx_a)
    pltpu.sync_copy(pd_hbm.at[pl.ds(wid * CH_W, CH_W)], idx_b)

    lane = lax.iota(jnp.int32, L)

    @pl.loop(0, CH_W)
    def _chunk(j):
        pltpu.async_copy(z_hbm.at[idx_a.at[j]], za, sem).wait()
        pltpu.async_copy(z_hbm.at[idx_b.at[j]], zb, sem).wait()

        @pl.loop(0, C // L)
        def _grp(g):
            vec = jnp.zeros((L,), jnp.float32)
            for k in range(L):
                e = g * L + k
                acc_v = za[e, pl.ds(0, L)] * zb[e, pl.ds(0, L)]
                for q in range(1, D // L):
                    acc_v = acc_v + za[e, pl.ds(q * L, L)] * zb[e, pl.ds(q * L, L)]
                tot = jnp.sum(acc_v)
                vec = jnp.where(lane == k, tot, vec)
            lg[j, pl.ds(g * L, L)] = vec

    pltpu.sync_copy(lg, out_hbm.at[pl.ds(wid * CH_W, CH_W)])


BN = 1000


def _mlp_body(x_ref, a_ref, w1_ref, b1_ref, w2_ref, b2_ref, z_ref):
    h = x_ref[...] + a_ref[0] + a_ref[1]
    hid = jnp.maximum(
        jnp.dot(h, w1_ref[...], preferred_element_type=jnp.float32) + b1_ref[...], 0.0
    )
    z_ref[...] = jnp.dot(hid, w2_ref[...], preferred_element_type=jnp.float32) + b2_ref[...]


def _mlp(x, aggp, W1, b1, W2, b2):
    return pl.pallas_call(
        _mlp_body,
        grid=(N // BN,),
        in_specs=[
            pl.BlockSpec((BN, D), lambda i: (i, 0)),
            pl.BlockSpec((NC, BN, D), lambda i: (0, i, 0)),
            pl.BlockSpec((D, D), lambda i: (0, 0)),
            pl.BlockSpec((1, D), lambda i: (0, 0)),
            pl.BlockSpec((D, D), lambda i: (0, 0)),
            pl.BlockSpec((1, D), lambda i: (0, 0)),
        ],
        out_specs=pl.BlockSpec((BN, D), lambda i: (i, 0)),
        out_shape=jax.ShapeDtypeStruct((N, D), jnp.float32),
    )(x, aggp, W1, b1.reshape(1, D), W2, b2.reshape(1, D))


@jax.jit
def kernel(x, edge_index, pos_edge_index, W1, b1, W2, b2):
    ei = edge_index.astype(jnp.int32)
    pe = pos_edge_index.astype(jnp.int32)
    src = ei[0].reshape(NCH, C)
    dst = ei[1].reshape(NCH, C)
    aggp = _agg_kernel(x, src, dst)
    z = _mlp(x, aggp, W1, b1, W2, b2)
    lg = _decode_kernel(z, pe[0].reshape(NCH, C), pe[1].reshape(NCH, C))
    return lg.reshape(E)


# speedup vs baseline: 5.0985x; 5.0985x over previous
import functools

import jax
import jax.numpy as jnp
from jax import lax
from jax.experimental import pallas as pl
from jax.experimental.pallas import tpu as pltpu
from jax.experimental.pallas import tpu_sc as plsc

N = 10000
D = 128
E = 320000

NC = 2
NS = 16
NW = NC * NS
L = 16

C = 80
E_W = E // NW
CH_W = E_W // C
ZB = 25

WB_START = 624
WB_ROWS = 632
WB_TAIL = N - (WB_START * (NS - 1) + WB_ROWS)

_mesh = plsc.VectorSubcoreMesh(
    core_axis_name="c", subcore_axis_name="s", num_cores=NC, num_subcores=NS
)


@functools.partial(
    pl.kernel,
    out_type=jax.ShapeDtypeStruct((NC, N, D), jnp.float32),
    mesh=_mesh,
    scratch_types=dict(
        acc=pltpu.VMEM_SHARED((N, D), jnp.float32),
        idx_s=pltpu.VMEM((E_W,), jnp.int32),
        idx_d=pltpu.VMEM((CH_W, C), jnp.int32),
        rows0=pltpu.VMEM((C, D), jnp.float32),
        rows1=pltpu.VMEM((C, D), jnp.float32),
        zbuf=pltpu.VMEM((ZB, D), jnp.float32),
        sem0=pltpu.SemaphoreType.DMA,
        sem1=pltpu.SemaphoreType.DMA,
        isem=pltpu.SemaphoreType.DMA,
    ),
)
def _agg_kernel(x_hbm, src_hbm, dst_hbm, out_hbm, *, acc, idx_s, idx_d, rows0, rows1,
                zbuf, sem0, sem1, isem):
    c = lax.axis_index("c")
    s = lax.axis_index("s")
    wid = s * NC + c

    zeros16 = jnp.zeros((L,), jnp.float32)

    @pl.loop(0, ZB)
    def _zero(i):
        for q in range(D // L):
            zbuf[i, pl.ds(q * L, L)] = zeros16

    pltpu.sync_copy(src_hbm.at[pl.ds(wid * E_W, E_W)], idx_s)
    descs = [
        pltpu.async_copy(dst_hbm.at[pl.ds(wid * E_W + j * C, C)], idx_d.at[j], isem)
        for j in range(CH_W)
    ]

    for r in range(625 // ZB):
        pltpu.sync_copy(zbuf, acc.at[pl.ds(s * 625 + r * ZB, ZB)])
    for d in descs:
        d.wait()
    plsc.subcore_barrier()

    slots = ((rows0, sem0), (rows1, sem1))

    def start(j, p):
        rows, sem = slots[p]
        pltpu.async_copy(x_hbm.at[idx_s.at[pl.ds(j * C, C)]], rows, sem)

    def wait(p):
        rows, sem = slots[p]
        pltpu.make_async_copy(x_hbm.at[pl.ds(0, C)], rows, sem).wait()

    def scat(j, p):
        rows, _ = slots[p]
        pltpu.sync_copy(rows, acc.at[idx_d.at[j]], add=True)

    start(0, 0)

    @pl.loop(0, (CH_W - 1) // 2)
    def _pair(i):
        for p in range(2):
            j = 2 * i + p
            wait(p)
            start(j + 1, 1 - p)
            scat(j, p)

    wait(0)
    scat(CH_W - 1, 0)

    plsc.subcore_barrier()
    pltpu.sync_copy(
        acc.at[pl.ds(s * WB_START, WB_ROWS)],
        out_hbm.at[c, pl.ds(s * WB_START, WB_ROWS)],
    )

    @pl.when(s == 0)
    def _tail():
        pltpu.sync_copy(
            acc.at[pl.ds(N - WB_TAIL, WB_TAIL)],
            out_hbm.at[c, pl.ds(N - WB_TAIL, WB_TAIL)],
        )


def _lane_sum_splat(v):
    lanes = lax.iota(jnp.int32, L)
    dnums = lax.GatherDimensionNumbers(
        offset_dims=(), collapsed_slice_dims=(0,), start_index_map=(0,)
    )
    for st in (8, 4, 2, 1):
        perm = lax.bitwise_xor(lanes, st)
        pv = lax.gather(
            v, perm[:, None], dnums, slice_sizes=(1,),
            mode=lax.GatherScatterMode.PROMISE_IN_BOUNDS,
        )
        v = v + pv
    return v


@functools.partial(
    pl.kernel,
    out_type=jax.ShapeDtypeStruct((E,), jnp.float32),
    mesh=_mesh,
    scratch_types=dict(
        idx_a=pltpu.VMEM((E_W,), jnp.int32),
        idx_b=pltpu.VMEM((E_W,), jnp.int32),
        za0=pltpu.VMEM((C, D), jnp.float32),
        zb0=pltpu.VMEM((C, D), jnp.float32),
        za1=pltpu.VMEM((C, D), jnp.float32),
        zb1=pltpu.VMEM((C, D), jnp.float32),
        lg=pltpu.VMEM((E_W,), jnp.float32),
        sem0=pltpu.SemaphoreType.DMA,
        sem1=pltpu.SemaphoreType.DMA,
    ),
)
def _decode_kernel(z_hbm, ps_hbm, pd_hbm, out_hbm, *, idx_a, idx_b, za0, zb0, za1,
                   zb1, lg, sem0, sem1):
    c = lax.axis_index("c")
    s = lax.axis_index("s")
    wid = s * NC + c

    pltpu.sync_copy(ps_hbm.at[pl.ds(wid * E_W, E_W)], idx_a)
    pltpu.sync_copy(pd_hbm.at[pl.ds(wid * E_W, E_W)], idx_b)

    lane = lax.iota(jnp.int32, L)
    slots = ((za0, zb0, sem0), (za1, zb1, sem1))

    def start(j, p):
        za, zb, sem = slots[p]
        pltpu.async_copy(z_hbm.at[idx_a.at[pl.ds(j * C, C)]], za, sem)
        pltpu.async_copy(z_hbm.at[idx_b.at[pl.ds(j * C, C)]], zb, sem)

    def wait(p):
        za, zb, sem = slots[p]
        pltpu.make_async_copy(z_hbm.at[pl.ds(0, C)], za, sem).wait()
        pltpu.make_async_copy(z_hbm.at[pl.ds(0, C)], zb, sem).wait()

    def compute(j, p):
        za, zb, _ = slots[p]
        for g in range(C // L):
            vec = jnp.zeros((L,), jnp.float32)
            for k in range(L):
                e = g * L + k
                acc_v = za[e, pl.ds(0, L)] * zb[e, pl.ds(0, L)]
                for q in range(1, D // L):
                    acc_v = acc_v + za[e, pl.ds(q * L, L)] * zb[e, pl.ds(q * L, L)]
                tot = _lane_sum_splat(acc_v)
                vec = jnp.where(lane == k, tot, vec)
            lg[pl.ds(j * C + g * L, L)] = vec

    start(0, 0)

    @pl.loop(0, (CH_W - 1) // 2)
    def _pair(i):
        for p in range(2):
            j = 2 * i + p
            wait(p)
            start(j + 1, 1 - p)
            compute(j, p)

    wait(0)
    compute(CH_W - 1, 0)

    pltpu.sync_copy(lg, out_hbm.at[pl.ds(wid * E_W, E_W)])


BN = 1000


def _mlp_body(x_ref, a_ref, w1_ref, b1_ref, w2_ref, b2_ref, z_ref):
    h = x_ref[...] + a_ref[0] + a_ref[1]
    hid = jnp.maximum(
        jnp.dot(h, w1_ref[...], preferred_element_type=jnp.float32) + b1_ref[...], 0.0
    )
    z_ref[...] = jnp.dot(hid, w2_ref[...], preferred_element_type=jnp.float32) + b2_ref[...]


def _mlp(x, aggp, W1, b1, W2, b2):
    return pl.pallas_call(
        _mlp_body,
        grid=(N // BN,),
        in_specs=[
            pl.BlockSpec((BN, D), lambda i: (i, 0)),
            pl.BlockSpec((NC, BN, D), lambda i: (0, i, 0)),
            pl.BlockSpec((D, D), lambda i: (0, 0)),
            pl.BlockSpec((1, D), lambda i: (0, 0)),
            pl.BlockSpec((D, D), lambda i: (0, 0)),
            pl.BlockSpec((1, D), lambda i: (0, 0)),
        ],
        out_specs=pl.BlockSpec((BN, D), lambda i: (i, 0)),
        out_shape=jax.ShapeDtypeStruct((N, D), jnp.float32),
    )(x, aggp, W1, b1.reshape(1, D), W2, b2.reshape(1, D))


@jax.jit
def kernel(x, edge_index, pos_edge_index, W1, b1, W2, b2):
    ei = edge_index.astype(jnp.int32)
    pe = pos_edge_index.astype(jnp.int32)
    aggp = _agg_kernel(x, ei[0], ei[1])
    z = _mlp(x, aggp, W1, b1, W2, b2)
    return _decode_kernel(z, pe[0], pe[1])


# speedup vs baseline: 6.0821x; 1.1929x over previous
import functools

import jax
import jax.numpy as jnp
from jax import lax
from jax.experimental import pallas as pl
from jax.experimental.pallas import tpu as pltpu
from jax.experimental.pallas import tpu_sc as plsc

N = 10000
D = 128
E = 320000

NC = 2
NS = 16
NW = NC * NS
L = 16

C = 80
E_W = E // NW
CH_W = E_W // C
ZB = 25

WB_START = 624
WB_ROWS = 632
WB_TAIL = N - (WB_START * (NS - 1) + WB_ROWS)

_mesh = plsc.VectorSubcoreMesh(
    core_axis_name="c", subcore_axis_name="s", num_cores=NC, num_subcores=NS
)


@functools.partial(
    pl.kernel,
    out_type=jax.ShapeDtypeStruct((NC, N, D), jnp.float32),
    mesh=_mesh,
    scratch_types=dict(
        acc=pltpu.VMEM_SHARED((N, D), jnp.float32),
        idx_s=pltpu.VMEM((E_W,), jnp.int32),
        idx_d=pltpu.VMEM((CH_W, C), jnp.int32),
        rows0=pltpu.VMEM((C, D), jnp.float32),
        rows1=pltpu.VMEM((C, D), jnp.float32),
        zbuf=pltpu.VMEM((ZB, D), jnp.float32),
        gsem0=pltpu.SemaphoreType.DMA,
        gsem1=pltpu.SemaphoreType.DMA,
        ssem0=pltpu.SemaphoreType.DMA,
        ssem1=pltpu.SemaphoreType.DMA,
        isem=pltpu.SemaphoreType.DMA,
    ),
)
def _agg_kernel(x_hbm, src_hbm, dst_hbm, out_hbm, *, acc, idx_s, idx_d, rows0, rows1,
                zbuf, gsem0, gsem1, ssem0, ssem1, isem):
    c = lax.axis_index("c")
    s = lax.axis_index("s")
    wid = s * NC + c

    zeros16 = jnp.zeros((L,), jnp.float32)

    @pl.loop(0, ZB)
    def _zero(i):
        for q in range(D // L):
            zbuf[i, pl.ds(q * L, L)] = zeros16

    pltpu.sync_copy(src_hbm.at[pl.ds(wid * E_W, E_W)], idx_s)
    descs = [
        pltpu.async_copy(dst_hbm.at[pl.ds(wid * E_W + j * C, C)], idx_d.at[j], isem)
        for j in range(CH_W)
    ]

    for r in range(625 // ZB):
        pltpu.sync_copy(zbuf, acc.at[pl.ds(s * 625 + r * ZB, ZB)])
    for d in descs:
        d.wait()
    plsc.subcore_barrier()

    rows = (rows0, rows1)
    gsem = (gsem0, gsem1)
    ssem = (ssem0, ssem1)

    def start_g(j, p):
        pltpu.async_copy(x_hbm.at[idx_s.at[pl.ds(j * C, C)]], rows[p], gsem[p])

    def wait_g(p):
        pltpu.make_async_copy(x_hbm.at[pl.ds(0, C)], rows[p], gsem[p]).wait()

    def start_s(j, p):
        pltpu.async_copy(rows[p], acc.at[idx_d.at[j]], ssem[p], add=True)

    def wait_s(p):
        pltpu.make_async_copy(x_hbm.at[pl.ds(0, C)], rows[p], ssem[p]).wait()

    start_g(0, 0)
    wait_g(0)
    start_s(0, 0)
    start_g(1, 1)

    @pl.loop(0, (CH_W - 3) // 2)
    def _pair(i):
        for p in range(2):
            j = 2 * i + 1 + p
            q = (1, 0)[p]
            wait_g(q)
            start_s(j, q)
            wait_s(p)
            start_g(j + 1, p)

    wait_g(1)
    start_s(CH_W - 2, 1)
    wait_s(0)
    start_g(CH_W - 1, 0)
    wait_g(0)
    start_s(CH_W - 1, 0)
    wait_s(1)
    wait_s(0)

    plsc.subcore_barrier()
    pltpu.sync_copy(
        acc.at[pl.ds(s * WB_START, WB_ROWS)],
        out_hbm.at[c, pl.ds(s * WB_START, WB_ROWS)],
    )

    @pl.when(s == 0)
    def _tail():
        pltpu.sync_copy(
            acc.at[pl.ds(N - WB_TAIL, WB_TAIL)],
            out_hbm.at[c, pl.ds(N - WB_TAIL, WB_TAIL)],
        )


def _lane_sum_splat(v):
    lanes = lax.iota(jnp.int32, L)
    dnums = lax.GatherDimensionNumbers(
        offset_dims=(), collapsed_slice_dims=(0,), start_index_map=(0,)
    )
    for st in (8, 4, 2, 1):
        perm = lax.bitwise_xor(lanes, st)
        pv = lax.gather(
            v, perm[:, None], dnums, slice_sizes=(1,),
            mode=lax.GatherScatterMode.PROMISE_IN_BOUNDS,
        )
        v = v + pv
    return v


@functools.partial(
    pl.kernel,
    out_type=jax.ShapeDtypeStruct((E,), jnp.float32),
    mesh=_mesh,
    scratch_types=dict(
        zsh=pltpu.VMEM_SHARED((N, D), jnp.float32),
        ia0=pltpu.VMEM((C,), jnp.int32),
        ia1=pltpu.VMEM((C,), jnp.int32),
        ib0=pltpu.VMEM((C,), jnp.int32),
        ib1=pltpu.VMEM((C,), jnp.int32),
        za0=pltpu.VMEM((C, D), jnp.float32),
        za1=pltpu.VMEM((C, D), jnp.float32),
        zb0=pltpu.VMEM((C, D), jnp.float32),
        zb1=pltpu.VMEM((C, D), jnp.float32),
        lg0=pltpu.VMEM((C,), jnp.float32),
        lg1=pltpu.VMEM((C,), jnp.float32),
        isem0=pltpu.SemaphoreType.DMA,
        isem1=pltpu.SemaphoreType.DMA,
        gsem0=pltpu.SemaphoreType.DMA,
        gsem1=pltpu.SemaphoreType.DMA,
        wsem0=pltpu.SemaphoreType.DMA,
        wsem1=pltpu.SemaphoreType.DMA,
    ),
)
def _decode_kernel(z_hbm, ps_hbm, pd_hbm, out_hbm, *, zsh, ia0, ia1, ib0, ib1,
                   za0, za1, zb0, zb1, lg0, lg1, isem0, isem1, gsem0, gsem1,
                   wsem0, wsem1):
    c = lax.axis_index("c")
    s = lax.axis_index("s")
    wid = s * NC + c
    base = wid * E_W

    pltpu.sync_copy(
        z_hbm.at[pl.ds(s * WB_START, WB_ROWS)], zsh.at[pl.ds(s * WB_START, WB_ROWS)]
    )

    @pl.when(s == 0)
    def _tail():
        pltpu.sync_copy(
            z_hbm.at[pl.ds(N - WB_TAIL, WB_TAIL)], zsh.at[pl.ds(N - WB_TAIL, WB_TAIL)]
        )

    plsc.subcore_barrier()

    ia = (ia0, ia1)
    ib = (ib0, ib1)
    za = (za0, za1)
    zb = (zb0, zb1)
    lg = (lg0, lg1)
    isem = (isem0, isem1)
    gsem = (gsem0, gsem1)
    wsem = (wsem0, wsem1)
    lane = lax.iota(jnp.int32, L)

    def start_idx(j, p):
        pltpu.async_copy(ps_hbm.at[pl.ds(base + j * C, C)], ia[p], isem[p])
        pltpu.async_copy(pd_hbm.at[pl.ds(base + j * C, C)], ib[p], isem[p])

    def wait_idx(p):
        pltpu.make_async_copy(ps_hbm.at[pl.ds(0, C)], ia[p], isem[p]).wait()
        pltpu.make_async_copy(ps_hbm.at[pl.ds(0, C)], ib[p], isem[p]).wait()

    def start_g(p):
        pltpu.async_copy(zsh.at[ia[p]], za[p], gsem[p])
        pltpu.async_copy(zsh.at[ib[p]], zb[p], gsem[p])

    def wait_g(p):
        pltpu.make_async_copy(z_hbm.at[pl.ds(0, C)], za[p], gsem[p]).wait()
        pltpu.make_async_copy(z_hbm.at[pl.ds(0, C)], zb[p], gsem[p]).wait()

    def start_w(j, p):
        pltpu.async_copy(lg[p], out_hbm.at[pl.ds(base + j * C, C)], wsem[p])

    def wait_w(p):
        pltpu.make_async_copy(out_hbm.at[pl.ds(0, C)], lg[p], wsem[p]).wait()

    def compute(p):
        zap, zbp, lgp = za[p], zb[p], lg[p]

        @pl.loop(0, C // L)
        def _grp(g):
            vec = jnp.zeros((L,), jnp.float32)
            for k in range(L):
                e = g * L + k
                acc_v = zap[e, pl.ds(0, L)] * zbp[e, pl.ds(0, L)]
                for q in range(1, D // L):
                    acc_v = acc_v + zap[e, pl.ds(q * L, L)] * zbp[e, pl.ds(q * L, L)]
                tot = _lane_sum_splat(acc_v)
                vec = jnp.where(lane == k, tot, vec)
            lgp[pl.ds(g * L, L)] = vec

    start_idx(0, 0)
    start_idx(1, 1)
    wait_idx(0)
    start_g(0)

    @pl.loop(0, (CH_W + 1) // 2)
    def _pair(i):
        for p in range(2):
            j = 2 * i + p

            @pl.when(j < CH_W)
            def _step():
                wait_g(p)

                @pl.when(j + 2 < CH_W)
                def _():
                    start_idx(j + 2, p)

                @pl.when(j + 1 < CH_W)
                def _():
                    wait_idx(1 - p)
                    start_g(1 - p)

                @pl.when(j >= 2)
                def _():
                    wait_w(p)

                compute(p)
                start_w(j, p)

    wait_w(1)
    wait_w(0)


BN = 1000


def _mlp_body(x_ref, a_ref, w1_ref, b1_ref, w2_ref, b2_ref, z_ref):
    h = x_ref[...] + a_ref[0] + a_ref[1]
    hid = jnp.maximum(
        jnp.dot(h, w1_ref[...], preferred_element_type=jnp.float32) + b1_ref[...], 0.0
    )
    z_ref[...] = jnp.dot(hid, w2_ref[...], preferred_element_type=jnp.float32) + b2_ref[...]


def _mlp(x, aggp, W1, b1, W2, b2):
    return pl.pallas_call(
        _mlp_body,
        grid=(N // BN,),
        in_specs=[
            pl.BlockSpec((BN, D), lambda i: (i, 0)),
            pl.BlockSpec((NC, BN, D), lambda i: (0, i, 0)),
            pl.BlockSpec((D, D), lambda i: (0, 0)),
            pl.BlockSpec((1, D), lambda i: (0, 0)),
            pl.BlockSpec((D, D), lambda i: (0, 0)),
            pl.BlockSpec((1, D), lambda i: (0, 0)),
        ],
        out_specs=pl.BlockSpec((BN, D), lambda i: (i, 0)),
        out_shape=jax.ShapeDtypeStruct((N, D), jnp.float32),
    )(x, aggp, W1, b1.reshape(1, D), W2, b2.reshape(1, D))


@jax.jit
def kernel(x, edge_index, pos_edge_index, W1, b1, W2, b2):
    ei = edge_index.astype(jnp.int32)
    pe = pos_edge_index.astype(jnp.int32)
    aggp = _agg_kernel(x, ei[0], ei[1])
    z = _mlp(x, aggp, W1, b1, W2, b2)
    return _decode_kernel(z, pe[0], pe[1])
